# Initial kernel scaffold; baseline (speedup 1.0000x reference)
import jax
import jax.numpy as jnp
from jax.experimental import pallas as pl


def kernel(params, target_features, form_features, role_features, form_neighbors, form_binds_ei, form_binds_y, form_binds_w, role_neighbors, role_binds_ei, role_binds_y, role_binds_w, form_diff_w, role_diff_w, trust_vector, drug_features):
    raise NotImplementedError("write your pallas kernel here")


# speedup vs baseline: 373.7163x; 373.7163x over previous
import functools
import math

import jax
import jax.numpy as jnp
from jax import lax
from jax.experimental import pallas as pl
from jax.experimental.pallas import tpu as pltpu
from jax.experimental.pallas import tpu_sc as plsc

P = 128
D = 128
N = 10000
E = 320000
NDRUG = 10000
BASELINE = 6.0

NP = 10240
GCH = NP // 16
ET = E // 16
CH = 2000
BIG = 1 << 30


def _leaky(x):
    return jnp.where(x >= 0, x, 0.2 * x)


def _prelu(x, a):
    return jnp.where(x >= 0, x, a * x)


def _lnorm(x, eps=1e-5):
    m = jnp.mean(x, axis=-1, keepdims=True)
    v = jnp.mean((x - m) * (x - m), axis=-1, keepdims=True)
    return (x - m) / jnp.sqrt(v + eps)


def _refine(x, w1T, b1, w2T, b2, alpha):
    h = jnp.dot(x, w1T[...], preferred_element_type=jnp.float32) + b1[...]
    h = _prelu(h, alpha[...])
    return jnp.dot(h, w2T[...], preferred_element_type=jnp.float32) + b2[...]


def _zt_of(t8, f1wT, f1b, f2wT, f2b, r1wT, r1b, r2wT, r2b, fpre, rpre):
    zf = _refine(t8[...], f1wT, f1b, f2wT, f2b, fpre)
    zr = _refine(t8[...], r1wT, r1b, r2wT, r2b, rpre)
    return 0.5 * (zf + zr)


def _tca1_body(t8, drugf,
               f1wT, f1b, f2wT, f2b, r1wT, r1b, r2wT, r2b, fpre, rpre,
               qwT, qb, kw, kb, vwT, vb,
               zt_o, ekq_o, V_o):
    zt = _zt_of(t8, f1wT, f1b, f2wT, f2b, r1wT, r1b, r2wT, r2b, fpre, rpre)
    q = jnp.dot(zt, qwT[...], preferred_element_type=jnp.float32) + qb[...]
    kqv = jnp.dot(q, kw[...], preferred_element_type=jnp.float32)
    kqb = jnp.sum(q[0:1] * kb[...], axis=1, keepdims=True)
    scale = 1.0 / math.sqrt(D)
    kq = (jnp.sum(drugf[...] * kqv[0:1], axis=1, keepdims=True) + kqb) * scale
    ekq_o[...] = jnp.exp(kq)
    V_o[...] = jnp.dot(drugf[...], vwT[...],
                       preferred_element_type=jnp.float32) + vb[...]
    zt_o[...] = zt[0:1]


def _tca2_body(t8, formf, rolef, fdw, rdw, lemb,
               f1wT, f1b, f2wT, f2b, r1wT, r1b, r2wT, r2b,
               a1ztT, a1nbT, a1embT, a1b, a2wT, a2b, a3wT, a3b,
               fpre, rpre,
               af_o, ar_o, df_o, dr_o, sums_o):
    zt = _zt_of(t8, f1wT, f1b, f2wT, f2b, r1wT, r1b, r2wT, r2b, fpre, rpre)
    fref = _refine(formf[...], f1wT, f1b, f2wT, f2b, fpre)
    rref = _refine(rolef[...], r1wT, r1b, r2wT, r2b, rpre)

    le = jnp.broadcast_to(lemb[0:1], (8, 16))
    embf = jnp.dot(le, a1embT[...], preferred_element_type=jnp.float32)
    le2 = jnp.broadcast_to(lemb[1:2], (8, 16))
    embr = jnp.dot(le2, a1embT[...], preferred_element_type=jnp.float32)
    ztc_f = (jnp.dot(zt, a1ztT[...], preferred_element_type=jnp.float32)
             + embf + a1b[...])[0:1]
    ztc_r = (jnp.dot(zt, a1ztT[...], preferred_element_type=jnp.float32)
             + embr + a1b[...])[0:1]

    def attn_mlp(ref_, ztc):
        a = jnp.dot(ref_, a1nbT[...], preferred_element_type=jnp.float32) + ztc
        a = _leaky(a)
        a = jnp.dot(a, a2wT[...], preferred_element_type=jnp.float32) + a2b[...]
        a = _leaky(a)
        return jnp.dot(a, a3wT[...], preferred_element_type=jnp.float32) + a3b[...]

    lgf = attn_mlp(fref, ztc_f) + jnp.log(jnp.clip(fdw[...], 1e-12))
    lgr = attn_mlp(rref, ztc_r) + jnp.log(jnp.clip(rdw[...], 1e-12))

    def smax(lg):
        m = jnp.max(lg, axis=0, keepdims=True)
        e = jnp.exp(lg - m)
        return e / jnp.sum(e, axis=0, keepdims=True)

    af = smax(lgf)
    ar = smax(lgr)
    af_o[...] = af
    ar_o[...] = ar
    df_o[...] = jnp.sum(af * fref, axis=0, keepdims=True)
    dr_o[...] = jnp.sum(ar * rref, axis=0, keepdims=True)
    sums_o[...] = jnp.concatenate(
        [jnp.sum(af, axis=0, keepdims=True), jnp.sum(ar, axis=0, keepdims=True)],
        axis=1)


def _sc_body(ei, ys, ws, nb, attn, ekq,
             r_out,
             nb_v, fn_v, ekq_v, attn_v, A_v, S_v, r_v,
             acc_v, tmp_v, g_buf, d_buf, y_buf, w_buf, shift_v,
             slots_sh, A_sh):
    c = lax.axis_index("c")
    s = lax.axis_index("s")
    base_g = s * GCH

    pltpu.sync_copy(ekq.at[pl.ds(0, N)], ekq_v.at[pl.ds(0, N)])
    pltpu.sync_copy(nb.at[pl.ds(c * N, N)], nb_v.at[pl.ds(0, N)])
    pltpu.sync_copy(attn.at[pl.ds(c * N, N)], attn_v.at[pl.ds(0, N)])

    zero16 = jnp.zeros((16,), jnp.float32)
    big16 = jnp.full((16,), BIG, jnp.int32)

    def init_body(i, _):
        fn_v[pl.ds(i * 16, 16)] = big16
        S_v[pl.ds(i * 16, 16)] = zero16
        r_v[pl.ds(i * 16, 16)] = zero16
        return 0
    lax.fori_loop(0, NP // 16, init_body, 0)

    shift_v[pl.ds(0, 16)] = jnp.full((16,), -1, jnp.int32)
    lane = lax.iota(jnp.int32, 16)

    def fn_body(k, _):
        b = (N // 16 - 1) - k
        base = b * 16
        g16 = nb_v[pl.ds(base, 16)]
        key = g16 * 16 + lane
        skey = lax.sort(key, dimension=0)
        gs = lax.shift_right_logical(skey, 4)
        is16 = base + (skey & 15)
        shift_v[pl.ds(1, 16)] = gs
        prev = shift_v[pl.ds(0, 16)]
        mask = gs != prev
        plsc.store_scatter(fn_v, [gs], is16, mask=mask)
        return 0
    lax.fori_loop(0, N // 16, fn_body, 0)

    def s_chunk(kc, _):
        base_e = s * ET + kc * CH
        pltpu.sync_copy(ei.at[pl.ds(c * 2 * E + base_e, CH)], g_buf)
        pltpu.sync_copy(ei.at[pl.ds(c * 2 * E + E + base_e, CH)], d_buf)

        def ebody(j, _):
            g16 = g_buf[pl.ds(j * 16, 16)]
            d16 = d_buf[pl.ds(j * 16, 16)]
            ek = plsc.load_gather(ekq_v, [d16])
            plsc.addupdate_scatter(S_v, [g16], ek)
            return 0
        lax.fori_loop(0, CH // 16, ebody, 0)
        return 0
    lax.fori_loop(0, ET // CH, s_chunk, 0)

    pltpu.sync_copy(S_v, slots_sh.at[s])
    plsc.subcore_barrier()
    pltpu.sync_copy(slots_sh.at[0, pl.ds(base_g, GCH)], acc_v)
    for t2 in range(1, 16):
        pltpu.sync_copy(slots_sh.at[t2, pl.ds(base_g, GCH)], tmp_v)

        def addb(j, _):
            acc_v[pl.ds(j * 16, 16)] = (acc_v[pl.ds(j * 16, 16)]
                                        + tmp_v[pl.ds(j * 16, 16)])
            return 0
        lax.fori_loop(0, GCH // 16, addb, 0)

    def abody(j, _):
        fn16 = fn_v[pl.ds(base_g + j * 16, 16)]
        valid = fn16 < N
        fnc = jnp.minimum(fn16, N - 1)
        a16 = plsc.load_gather(attn_v, [fnc])
        s16 = acc_v[pl.ds(j * 16, 16)]
        tmp_v[pl.ds(j * 16, 16)] = jnp.where(valid, a16 / (s16 + 1e-12), 0.0)
        return 0
    lax.fori_loop(0, GCH // 16, abody, 0)
    pltpu.sync_copy(tmp_v, A_sh.at[pl.ds(base_g, GCH)])
    plsc.subcore_barrier()
    pltpu.sync_copy(A_sh, A_v)

    def r_chunk(kc, _):
        base_e = s * ET + kc * CH
        pltpu.sync_copy(ei.at[pl.ds(c * 2 * E + base_e, CH)], g_buf)
        pltpu.sync_copy(ei.at[pl.ds(c * 2 * E + E + base_e, CH)], d_buf)
        pltpu.sync_copy(ys.at[pl.ds(c * E + base_e, CH)], y_buf)
        pltpu.sync_copy(ws.at[pl.ds(c * E + base_e, CH)], w_buf)

        def ebody(j, _):
            g16 = g_buf[pl.ds(j * 16, 16)]
            d16 = d_buf[pl.ds(j * 16, 16)]
            y16 = y_buf[pl.ds(j * 16, 16)]
            w16 = w_buf[pl.ds(j * 16, 16)]
            a = plsc.load_gather(A_v, [g16])
            ek = plsc.load_gather(ekq_v, [d16])
            u = a * ek * ((y16 - BASELINE) * w16)
            plsc.addupdate_scatter(r_v, [d16], u)
            return 0
        lax.fori_loop(0, CH // 16, ebody, 0)
        return 0
    lax.fori_loop(0, ET // CH, r_chunk, 0)

    pltpu.sync_copy(r_v, slots_sh.at[s])
    plsc.subcore_barrier()
    pltpu.sync_copy(slots_sh.at[0, pl.ds(base_g, GCH)], acc_v)
    for t2 in range(1, 16):
        pltpu.sync_copy(slots_sh.at[t2, pl.ds(base_g, GCH)], tmp_v)

        def addb2(j, _):
            acc_v[pl.ds(j * 16, 16)] = (acc_v[pl.ds(j * 16, 16)]
                                        + tmp_v[pl.ds(j * 16, 16)])
            return 0
        lax.fori_loop(0, GCH // 16, addb2, 0)
    pltpu.sync_copy(acc_v, r_out.at[pl.ds(c * NP + base_g, GCH)])


_SC_SCRATCH = [
        pltpu.VMEM((NP,), jnp.int32),
        pltpu.VMEM((NP,), jnp.int32),
        pltpu.VMEM((NP,), jnp.float32),
        pltpu.VMEM((NP,), jnp.float32),
        pltpu.VMEM((NP,), jnp.float32),
        pltpu.VMEM((NP,), jnp.float32),
        pltpu.VMEM((NP,), jnp.float32),
        pltpu.VMEM((GCH,), jnp.float32),
        pltpu.VMEM((GCH,), jnp.float32),
        pltpu.VMEM((CH,), jnp.int32),
        pltpu.VMEM((CH,), jnp.int32),
        pltpu.VMEM((CH,), jnp.float32),
        pltpu.VMEM((CH,), jnp.float32),
        pltpu.VMEM((32,), jnp.int32),
        pltpu.VMEM_SHARED((16, NP), jnp.float32),
        pltpu.VMEM_SHARED((NP,), jnp.float32),
    ]


@functools.lru_cache(maxsize=None)
def _get_sc_edges():
    return functools.partial(
        pl.kernel,
        out_type=jax.ShapeDtypeStruct((2 * NP,), jnp.float32),
        mesh=plsc.VectorSubcoreMesh(core_axis_name="c", subcore_axis_name="s",
                                    num_cores=2, num_subcores=16),
        scratch_types=_SC_SCRATCH,
        compiler_params=pltpu.CompilerParams(needs_layout_passes=False),
    )(_sc_body)


def _tcb_body(rf, rr, V, zt, df, dr, sums, trust8,
              m1vfT, m1vrT, m1tT, m1b, m2wT, m2b,
              i1wT, i1b, i2wT, i2b, normw, normb,
              mixpre, integpre, dgate,
              zref_o, vprior_o, dmean_o):
    vf1 = jnp.sum(rf[...] * V[...], axis=0, keepdims=True)
    vr1 = jnp.sum(rr[...] * V[...], axis=0, keepdims=True)
    vf = jnp.broadcast_to(vf1, (8, D))
    vr = jnp.broadcast_to(vr1, (8, D))
    h = (jnp.dot(vf, m1vfT[...], preferred_element_type=jnp.float32)
         + jnp.dot(vr, m1vrT[...], preferred_element_type=jnp.float32)
         + jnp.dot(trust8[...], m1tT[...], preferred_element_type=jnp.float32)
         + m1b[...])
    h = _prelu(h, mixpre[...])
    l2 = jnp.dot(h, m2wT[...], preferred_element_type=jnp.float32) + m2b[...]
    m = jnp.max(l2, axis=1, keepdims=True)
    e2 = jnp.exp(l2 - m)
    lw = e2 / jnp.sum(e2, axis=1, keepdims=True)
    wf = lw[:, 0:1]
    wr = lw[:, 1:2]
    vprior = wf * vf + wr * vr
    zt8 = jnp.broadcast_to(zt[...], (8, P))
    fdelta = zt8 * sums[0:1, 0:1] - jnp.broadcast_to(df[...], (8, P))
    rdelta = zt8 * sums[0:1, 1:2] - jnp.broadcast_to(dr[...], (8, P))
    draw = wf * fdelta + wr * rdelta
    dmean = dgate[...] * _lnorm(draw)
    h2 = _prelu(jnp.dot(vprior, i1wT[...], preferred_element_type=jnp.float32)
                + i1b[...], integpre[...])
    zref = (_lnorm(zt8 + jnp.dot(h2, i2wT[...],
                                 preferred_element_type=jnp.float32) + i2b[...])
            * normw[...] + normb[...])
    zref_o[...] = zref[0:1]
    vprior_o[...] = vprior[0:1]
    dmean_o[...] = dmean[0:1]


def kernel(params, target_features, form_features, role_features,
           form_neighbors, form_binds_ei, form_binds_y, form_binds_w,
           role_neighbors, role_binds_ei, role_binds_y, role_binds_w,
           form_diff_w, role_diff_w, trust_vector, drug_features):
    p = params
    f32 = jnp.float32

    t8 = jnp.tile(target_features[None, :].astype(f32), (8, 1))
    fdw = form_diff_w[:, None]
    rdw = role_diff_w[:, None]
    a1w = p['attn1_w']

    refine_ws = (
        p['form1_w'].T, p['form1_b'][None], p['form2_w'].T, p['form2_b'][None],
        p['role1_w'].T, p['role1_b'][None], p['role2_w'].T, p['role2_b'][None],
    )
    preluses = (p['form_prelu'][None, None], p['role_prelu'][None, None])

    zt, ekq, V = pl.pallas_call(
        _tca1_body,
        out_shape=[
            jax.ShapeDtypeStruct((1, P), f32),
            jax.ShapeDtypeStruct((NDRUG, 1), f32),
            jax.ShapeDtypeStruct((NDRUG, D), f32),
        ],
    )(
        t8, drug_features, *refine_ws, *preluses,
        p['q_w'].T, p['q_b'][None], p['k_w'], p['k_b'][None],
        p['v_w'].T, p['v_b'][None],
    )

    af, ar, df, dr, sums = pl.pallas_call(
        _tca2_body,
        out_shape=[
            jax.ShapeDtypeStruct((N, 1), f32),
            jax.ShapeDtypeStruct((N, 1), f32),
            jax.ShapeDtypeStruct((1, P), f32),
            jax.ShapeDtypeStruct((1, P), f32),
            jax.ShapeDtypeStruct((1, 2), f32),
        ],
    )(
        t8, form_features, role_features, fdw, rdw, p['layer_emb'],
        *refine_ws,
        a1w[:, :P].T, a1w[:, P:2 * P].T, a1w[:, 2 * P:].T, p['attn1_b'][None],
        p['attn2_w'].T, p['attn2_b'][None], p['attn3_w'].T, p['attn3_b'][None],
        *preluses,
    )

    ei_flat = jnp.concatenate(
        [form_binds_ei.reshape(-1), role_binds_ei.reshape(-1)]
    ).astype(jnp.int32)
    y_flat = jnp.concatenate([form_binds_y, role_binds_y])
    w_flat = jnp.concatenate([form_binds_w, role_binds_w])
    nb_flat = jnp.concatenate([form_neighbors, role_neighbors]).astype(jnp.int32)
    attn_flat = jnp.concatenate([af[:, 0], ar[:, 0]])

    r = _get_sc_edges()(ei_flat, y_flat, w_flat, nb_flat, attn_flat, ekq[:, 0])

    zref, vprior, dmean = pl.pallas_call(
        _tcb_body,
        out_shape=[
            jax.ShapeDtypeStruct((1, P), f32),
            jax.ShapeDtypeStruct((1, P), f32),
            jax.ShapeDtypeStruct((1, P), f32),
        ],
    )(
        r[:N, None], r[NP:NP + N, None], V, zt, df, dr, sums,
        jnp.tile(trust_vector[None, :].astype(f32), (8, 1)),
        p['mix1_w'][:, :D].T, p['mix1_w'][:, D:2 * D].T,
        p['mix1_w'][:, 2 * D:].T, p['mix1_b'][None],
        p['mix2_w'].T, p['mix2_b'][None],
        p['integ1_w'].T, p['integ1_b'][None],
        p['integ2_w'].T, p['integ2_b'][None],
        p['norm_w'][None], p['norm_b'][None],
        p['mix_prelu'][None, None], p['integ_prelu'][None, None],
        p['delta_gate'][None, None],
    )

    return (zref[0], vprior[0], dmean[0], af[:, 0], ar[:, 0])


# speedup vs baseline: 449.9067x; 1.2039x over previous
import functools
import math

import jax
import jax.numpy as jnp
from jax import lax
from jax.experimental import pallas as pl
from jax.experimental.pallas import tpu as pltpu
from jax.experimental.pallas import tpu_sc as plsc

P = 128
D = 128
N = 10000
E = 320000
NDRUG = 10000
BASELINE = 6.0

NP = 10240
GCH = NP // 16
ET = E // 16
CH = 2000
BIG = 1 << 30


def _leaky(x):
    return jnp.where(x >= 0, x, 0.2 * x)


def _prelu(x, a):
    return jnp.where(x >= 0, x, a * x)


def _lnorm(x, eps=1e-5):
    m = jnp.mean(x, axis=-1, keepdims=True)
    v = jnp.mean((x - m) * (x - m), axis=-1, keepdims=True)
    return (x - m) / jnp.sqrt(v + eps)


def _refine(x, w1T, b1, w2T, b2, alpha):
    h = jnp.dot(x, w1T[...], preferred_element_type=jnp.float32) + b1[...]
    h = _prelu(h, alpha[...])
    return jnp.dot(h, w2T[...], preferred_element_type=jnp.float32) + b2[...]


def _zt_of(t8, f1wT, f1b, f2wT, f2b, r1wT, r1b, r2wT, r2b, fpre, rpre):
    zf = _refine(t8[...], f1wT, f1b, f2wT, f2b, fpre)
    zr = _refine(t8[...], r1wT, r1b, r2wT, r2b, rpre)
    return 0.5 * (zf + zr)


def _tca1_body(t8, drugf, yf2, wf2, yr2, wr2,
               f1wT, f1b, f2wT, f2b, r1wT, r1b, r2wT, r2b, fpre, rpre,
               qwT, qb, kw, kb, vwT, vb,
               zt_o, ekq_o, V_o, sc_o):
    sc_o[0:E // 128] = (yf2[...] - BASELINE) * wf2[...]
    sc_o[E // 128:2 * (E // 128)] = (yr2[...] - BASELINE) * wr2[...]
    zt = _zt_of(t8, f1wT, f1b, f2wT, f2b, r1wT, r1b, r2wT, r2b, fpre, rpre)
    q = jnp.dot(zt, qwT[...], preferred_element_type=jnp.float32) + qb[...]
    kqv = jnp.dot(q, kw[...], preferred_element_type=jnp.float32)
    kqb = jnp.sum(q[0:1] * kb[...], axis=1, keepdims=True)
    scale = 1.0 / math.sqrt(D)
    kq = (jnp.sum(drugf[...] * kqv[0:1], axis=1, keepdims=True) + kqb) * scale
    ekq_o[...] = jnp.exp(kq)
    V_o[...] = jnp.dot(drugf[...], vwT[...],
                       preferred_element_type=jnp.float32) + vb[...]
    zt_o[...] = zt[0:1]


def _tca2_body(t8, formf, rolef, fdw, rdw, lemb,
               f1wT, f1b, f2wT, f2b, r1wT, r1b, r2wT, r2b,
               a1ztT, a1nbT, a1embT, a1b, a2wT, a2b, a3wT, a3b,
               fpre, rpre,
               at2_o, df_o, dr_o, sums_o):
    zt = _zt_of(t8, f1wT, f1b, f2wT, f2b, r1wT, r1b, r2wT, r2b, fpre, rpre)
    fref = _refine(formf[...], f1wT, f1b, f2wT, f2b, fpre)
    rref = _refine(rolef[...], r1wT, r1b, r2wT, r2b, rpre)

    le = jnp.broadcast_to(lemb[0:1], (8, 16))
    embf = jnp.dot(le, a1embT[...], preferred_element_type=jnp.float32)
    le2 = jnp.broadcast_to(lemb[1:2], (8, 16))
    embr = jnp.dot(le2, a1embT[...], preferred_element_type=jnp.float32)
    ztc_f = (jnp.dot(zt, a1ztT[...], preferred_element_type=jnp.float32)
             + embf + a1b[...])[0:1]
    ztc_r = (jnp.dot(zt, a1ztT[...], preferred_element_type=jnp.float32)
             + embr + a1b[...])[0:1]

    def attn_mlp(ref_, ztc):
        a = jnp.dot(ref_, a1nbT[...], preferred_element_type=jnp.float32) + ztc
        a = _leaky(a)
        a = jnp.dot(a, a2wT[...], preferred_element_type=jnp.float32) + a2b[...]
        a = _leaky(a)
        return jnp.dot(a, a3wT[...], preferred_element_type=jnp.float32) + a3b[...]

    lgf = attn_mlp(fref, ztc_f) + jnp.log(jnp.clip(fdw[...], 1e-12))
    lgr = attn_mlp(rref, ztc_r) + jnp.log(jnp.clip(rdw[...], 1e-12))

    def smax(lg):
        m = jnp.max(lg, axis=0, keepdims=True)
        e = jnp.exp(lg - m)
        return e / jnp.sum(e, axis=0, keepdims=True)

    af = smax(lgf)
    ar = smax(lgr)
    at2_o[0:N] = af
    at2_o[N:2 * N] = ar
    df_o[...] = jnp.sum(af * fref, axis=0, keepdims=True)
    dr_o[...] = jnp.sum(ar * rref, axis=0, keepdims=True)
    sums_o[...] = jnp.concatenate(
        [jnp.sum(af, axis=0, keepdims=True), jnp.sum(ar, axis=0, keepdims=True)],
        axis=1)


UN = 5


def _sc_body(ei, sc, nb, attn, ekq,
             r_out,
             nb_v, fn_v, ekq_v, attn_v, S_v,
             acc_v, tmp_v, g_all, d_all, sc_all, shift_v, sem_a, sem_b,
             slots_sh, A_sh):
    c = lax.axis_index("c")
    s = lax.axis_index("s")
    base_g = s * GCH
    base_e = s * ET
    r_v = S_v

    copies_a = [
        (ekq.at[pl.ds(0, N)], ekq_v.at[pl.ds(0, N)]),
        (nb.at[pl.ds(c * N, N)], nb_v.at[pl.ds(0, N)]),
        (attn.at[pl.ds(c * N, N)], attn_v.at[pl.ds(0, N)]),
    ]
    copies_b = [
        (ei.at[pl.ds(c * 2 * E + base_e, ET)], g_all),
        (ei.at[pl.ds(c * 2 * E + E + base_e, ET)], d_all),
        (sc.at[pl.ds(c * E + base_e, ET)], sc_all),
    ]
    descs_a = [pltpu.async_copy(s_, d_, sem_a) for s_, d_ in copies_a]
    descs_b = [pltpu.async_copy(s_, d_, sem_b) for s_, d_ in copies_b]

    zero16 = jnp.zeros((16,), jnp.float32)
    big16 = jnp.full((16,), BIG, jnp.int32)

    def init_body(i, _):
        for u in range(8):
            fn_v[pl.ds(i * 128 + u * 16, 16)] = big16
            S_v[pl.ds(i * 128 + u * 16, 16)] = zero16
        return 0
    lax.fori_loop(0, NP // 128, init_body, 0)

    for u in range(UN):
        shift_v[pl.ds(u * 32, 16)] = jnp.full((16,), -1, jnp.int32)
    lane = lax.iota(jnp.int32, 16)

    for d_ in descs_a:
        d_.wait()

    def fn_body(k, _):
        for u in range(UN):
            b = (N // 16 - 1) - (k * UN + u)
            base = b * 16
            g16 = nb_v[pl.ds(base, 16)]
            key = g16 * 16 + lane
            skey = lax.sort(key, dimension=0)
            gs = lax.shift_right_logical(skey, 4)
            is16 = base + (skey & 15)
            shift_v[pl.ds(u * 32 + 1, 16)] = gs
            prev = shift_v[pl.ds(u * 32, 16)]
            mask = gs != prev
            plsc.store_scatter(fn_v, [gs], is16, mask=mask)
        return 0
    lax.fori_loop(0, N // 16 // UN, fn_body, 0)

    for d_ in descs_b:
        d_.wait()

    def s_group(j, _):
        for u in range(UN):
            off = j * (16 * UN) + u * 16
            g16 = g_all[pl.ds(off, 16)]
            d16 = d_all[pl.ds(off, 16)]
            ek = plsc.load_gather(ekq_v, [d16])
            plsc.addupdate_scatter(S_v, [g16], ek)
        return 0
    lax.fori_loop(0, ET // (16 * UN), s_group, 0)

    pltpu.sync_copy(S_v, slots_sh.at[s])

    def zero_body(i, _):
        for u in range(8):
            r_v[pl.ds(i * 128 + u * 16, 16)] = zero16
        return 0
    lax.fori_loop(0, NP // 128, zero_body, 0)

    plsc.subcore_barrier()
    pltpu.sync_copy(slots_sh.at[0, pl.ds(base_g, GCH)], acc_v)
    for t2 in range(1, 16):
        pltpu.sync_copy(slots_sh.at[t2, pl.ds(base_g, GCH)], tmp_v)

        def addb(j, _):
            for u in range(8):
                o = j * 128 + u * 16
                acc_v[pl.ds(o, 16)] = acc_v[pl.ds(o, 16)] + tmp_v[pl.ds(o, 16)]
            return 0
        lax.fori_loop(0, GCH // 128, addb, 0)

    def abody(j, _):
        for u in range(8):
            o = j * 128 + u * 16
            fn16 = fn_v[pl.ds(base_g + o, 16)]
            valid = fn16 < N
            fnc = jnp.minimum(fn16, N - 1)
            a16 = plsc.load_gather(attn_v, [fnc])
            s16 = acc_v[pl.ds(o, 16)]
            tmp_v[pl.ds(o, 16)] = jnp.where(valid, a16 / (s16 + 1e-12), 0.0)
        return 0
    lax.fori_loop(0, GCH // 128, abody, 0)
    pltpu.sync_copy(tmp_v, A_sh.at[pl.ds(base_g, GCH)])
    plsc.subcore_barrier()
    A_v = attn_v
    pltpu.sync_copy(A_sh, A_v)

    def r_group(j, _):
        for u in range(UN):
            off = j * (16 * UN) + u * 16
            g16 = g_all[pl.ds(off, 16)]
            d16 = d_all[pl.ds(off, 16)]
            s16 = sc_all[pl.ds(off, 16)]
            a = plsc.load_gather(A_v, [g16])
            ek = plsc.load_gather(ekq_v, [d16])
            plsc.addupdate_scatter(r_v, [d16], a * ek * s16)
        return 0
    lax.fori_loop(0, ET // (16 * UN), r_group, 0)

    pltpu.sync_copy(r_v, slots_sh.at[s])
    plsc.subcore_barrier()
    pltpu.sync_copy(slots_sh.at[0, pl.ds(base_g, GCH)], acc_v)
    for t2 in range(1, 16):
        pltpu.sync_copy(slots_sh.at[t2, pl.ds(base_g, GCH)], tmp_v)

        def addb2(j, _):
            for u in range(8):
                o = j * 128 + u * 16
                acc_v[pl.ds(o, 16)] = acc_v[pl.ds(o, 16)] + tmp_v[pl.ds(o, 16)]
            return 0
        lax.fori_loop(0, GCH // 128, addb2, 0)
    pltpu.sync_copy(acc_v, r_out.at[pl.ds(c * NP + base_g, GCH)])


_SC_SCRATCH = [
        pltpu.VMEM((NP,), jnp.int32),
        pltpu.VMEM((NP,), jnp.int32),
        pltpu.VMEM((NP,), jnp.float32),
        pltpu.VMEM((NP,), jnp.float32),
        pltpu.VMEM((NP,), jnp.float32),
        pltpu.VMEM((GCH,), jnp.float32),
        pltpu.VMEM((GCH,), jnp.float32),
        pltpu.VMEM((ET,), jnp.int32),
        pltpu.VMEM((ET,), jnp.int32),
        pltpu.VMEM((ET,), jnp.float32),
        pltpu.VMEM((UN * 32,), jnp.int32),
        pltpu.SemaphoreType.DMA,
        pltpu.SemaphoreType.DMA,
        pltpu.VMEM_SHARED((16, NP), jnp.float32),
        pltpu.VMEM_SHARED((NP,), jnp.float32),
    ]


@functools.lru_cache(maxsize=None)
def _get_sc_edges():
    return functools.partial(
        pl.kernel,
        out_type=jax.ShapeDtypeStruct((2 * NP,), jnp.float32),
        mesh=plsc.VectorSubcoreMesh(core_axis_name="c", subcore_axis_name="s",
                                    num_cores=2, num_subcores=16),
        scratch_types=_SC_SCRATCH,
        compiler_params=pltpu.CompilerParams(needs_layout_passes=False),
    )(_sc_body)


def _tcb_body(rall, V, zt, df, dr, sums, trust8,
              m1vfT, m1vrT, m1tT, m1b, m2wT, m2b,
              i1wT, i1b, i2wT, i2b, normw, normb,
              mixpre, integpre, dgate,
              zref_o, vprior_o, dmean_o):
    rf = rall[0:N, :]
    rr = rall[NP:NP + N, :]
    vf1 = jnp.sum(rf * V[...], axis=0, keepdims=True)
    vr1 = jnp.sum(rr * V[...], axis=0, keepdims=True)
    vf = jnp.broadcast_to(vf1, (8, D))
    vr = jnp.broadcast_to(vr1, (8, D))
    h = (jnp.dot(vf, m1vfT[...], preferred_element_type=jnp.float32)
         + jnp.dot(vr, m1vrT[...], preferred_element_type=jnp.float32)
         + jnp.dot(trust8[...], m1tT[...], preferred_element_type=jnp.float32)
         + m1b[...])
    h = _prelu(h, mixpre[...])
    l2 = jnp.dot(h, m2wT[...], preferred_element_type=jnp.float32) + m2b[...]
    m = jnp.max(l2, axis=1, keepdims=True)
    e2 = jnp.exp(l2 - m)
    lw = e2 / jnp.sum(e2, axis=1, keepdims=True)
    wf = lw[:, 0:1]
    wr = lw[:, 1:2]
    vprior = wf * vf + wr * vr
    zt8 = jnp.broadcast_to(zt[...], (8, P))
    fdelta = zt8 * sums[0:1, 0:1] - jnp.broadcast_to(df[...], (8, P))
    rdelta = zt8 * sums[0:1, 1:2] - jnp.broadcast_to(dr[...], (8, P))
    draw = wf * fdelta + wr * rdelta
    dmean = dgate[...] * _lnorm(draw)
    h2 = _prelu(jnp.dot(vprior, i1wT[...], preferred_element_type=jnp.float32)
                + i1b[...], integpre[...])
    zref = (_lnorm(zt8 + jnp.dot(h2, i2wT[...],
                                 preferred_element_type=jnp.float32) + i2b[...])
            * normw[...] + normb[...])
    zref_o[...] = zref[0:1]
    vprior_o[...] = vprior[0:1]
    dmean_o[...] = dmean[0:1]


def kernel(params, target_features, form_features, role_features,
           form_neighbors, form_binds_ei, form_binds_y, form_binds_w,
           role_neighbors, role_binds_ei, role_binds_y, role_binds_w,
           form_diff_w, role_diff_w, trust_vector, drug_features):
    p = params
    f32 = jnp.float32

    t8 = jnp.tile(target_features[None, :].astype(f32), (8, 1))
    fdw = form_diff_w[:, None]
    rdw = role_diff_w[:, None]
    a1w = p['attn1_w']

    refine_ws = (
        p['form1_w'].T, p['form1_b'][None], p['form2_w'].T, p['form2_b'][None],
        p['role1_w'].T, p['role1_b'][None], p['role2_w'].T, p['role2_b'][None],
    )
    preluses = (p['form_prelu'][None, None], p['role_prelu'][None, None])

    zt, ekq, V, sc2 = pl.pallas_call(
        _tca1_body,
        out_shape=[
            jax.ShapeDtypeStruct((1, P), f32),
            jax.ShapeDtypeStruct((NDRUG, 1), f32),
            jax.ShapeDtypeStruct((NDRUG, D), f32),
            jax.ShapeDtypeStruct((2 * (E // 128), 128), f32),
        ],
    )(
        t8, drug_features,
        form_binds_y.reshape(E // 128, 128), form_binds_w.reshape(E // 128, 128),
        role_binds_y.reshape(E // 128, 128), role_binds_w.reshape(E // 128, 128),
        *refine_ws, *preluses,
        p['q_w'].T, p['q_b'][None], p['k_w'], p['k_b'][None],
        p['v_w'].T, p['v_b'][None],
    )

    at2, df, dr, sums = pl.pallas_call(
        _tca2_body,
        out_shape=[
            jax.ShapeDtypeStruct((2 * N, 1), f32),
            jax.ShapeDtypeStruct((1, P), f32),
            jax.ShapeDtypeStruct((1, P), f32),
            jax.ShapeDtypeStruct((1, 2), f32),
        ],
    )(
        t8, form_features, role_features, fdw, rdw, p['layer_emb'],
        *refine_ws,
        a1w[:, :P].T, a1w[:, P:2 * P].T, a1w[:, 2 * P:].T, p['attn1_b'][None],
        p['attn2_w'].T, p['attn2_b'][None], p['attn3_w'].T, p['attn3_b'][None],
        *preluses,
    )

    ei_all = jnp.concatenate([form_binds_ei.reshape(-1),
                              role_binds_ei.reshape(-1)]).astype(jnp.int32)
    nb_all = jnp.concatenate([form_neighbors,
                              role_neighbors]).astype(jnp.int32)
    r = _get_sc_edges()(ei_all, sc2.reshape(-1), nb_all, at2.reshape(-1),
                        ekq.reshape(-1))

    zref, vprior, dmean = pl.pallas_call(
        _tcb_body,
        out_shape=[
            jax.ShapeDtypeStruct((1, P), f32),
            jax.ShapeDtypeStruct((1, P), f32),
            jax.ShapeDtypeStruct((1, P), f32),
        ],
    )(
        r[:, None], V, zt, df, dr, sums,
        jnp.tile(trust_vector[None, :].astype(f32), (8, 1)),
        p['mix1_w'][:, :D].T, p['mix1_w'][:, D:2 * D].T,
        p['mix1_w'][:, 2 * D:].T, p['mix1_b'][None],
        p['mix2_w'].T, p['mix2_b'][None],
        p['integ1_w'].T, p['integ1_b'][None],
        p['integ2_w'].T, p['integ2_b'][None],
        p['norm_w'][None], p['norm_b'][None],
        p['mix_prelu'][None, None], p['integ_prelu'][None, None],
        p['delta_gate'][None, None],
    )

    return (zref[0], vprior[0], dmean[0], at2[:N, 0], at2[N:, 0])


# speedup vs baseline: 701.4714x; 1.5591x over previous
import functools
import math

import jax
import jax.numpy as jnp
from jax import lax
from jax.experimental import pallas as pl
from jax.experimental.pallas import tpu as pltpu
from jax.experimental.pallas import tpu_sc as plsc

P = 128
D = 128
N = 10000
E = 320000
NDRUG = 10000
BASELINE = 6.0

NP = 10240
GCH = NP // 16
ET = E // 16
CH = 2000
BIG = 1 << 30


def _leaky(x):
    return jnp.where(x >= 0, x, 0.2 * x)


def _prelu(x, a):
    return jnp.where(x >= 0, x, a * x)


def _lnorm(x, eps=1e-5):
    m = jnp.mean(x, axis=-1, keepdims=True)
    v = jnp.mean((x - m) * (x - m), axis=-1, keepdims=True)
    return (x - m) / jnp.sqrt(v + eps)


def _matT(x, w):
    return lax.dot_general(x, w, (((1,), (1,)), ((), ())),
                           preferred_element_type=jnp.float32)


def _refine(x, w1, b1, w2, b2, alpha):
    h = _matT(x, w1[...]) + b1[...]
    h = _prelu(h, alpha[...])
    return _matT(h, w2[...]) + b2[...]


def _zt_of(t8, f1w, f1b, f2w, f2b, r1w, r1b, r2w, r2b, fpre, rpre):
    zf = _refine(t8[...], f1w, f1b, f2w, f2b, fpre)
    zr = _refine(t8[...], r1w, r1b, r2w, r2b, rpre)
    return 0.5 * (zf + zr)


def _tca1_body(t8, drugf, yf, wf, yr, wr,
               f1w, f1b, f2w, f2b, r1w, r1b, r2w, r2b, fpre, rpre,
               qw, qb, kw, kb, vw, vb,
               zt_o, ekq_o, V_o, sc_o):
    sc_o[0:E] = (yf[...] - BASELINE) * wf[...]
    sc_o[E:2 * E] = (yr[...] - BASELINE) * wr[...]
    zt = _zt_of(t8, f1w, f1b, f2w, f2b, r1w, r1b, r2w, r2b, fpre, rpre)
    q = _matT(zt, qw[...]) + qb[...]
    kqv = jnp.dot(q, kw[...], preferred_element_type=jnp.float32)
    kqb = jnp.sum(q[0:1] * kb[...], axis=1, keepdims=True)
    scale = 1.0 / math.sqrt(D)
    kq_row = lax.dot_general(kqv, drugf[...], (((1,), (1,)), ((), ())),
                             preferred_element_type=jnp.float32)
    ekq_o[...] = jnp.reshape(jnp.exp((kq_row[0:1] + kqb) * scale), (NDRUG,))
    V_o[...] = _matT(drugf[...], vw[...]) + vb[...]
    zt_o[...] = zt[0:1]


def _tca2_body(t8, formf, rolef, fdw, rdw, lemb,
               f1w, f1b, f2w, f2b, r1w, r1b, r2w, r2b,
               a1w, a1b, a2w, a2b, a3w, a3b,
               fpre, rpre,
               at2_o, df_o, dr_o, sums_o):
    zt = _zt_of(t8, f1w, f1b, f2w, f2b, r1w, r1b, r2w, r2b, fpre, rpre)
    fref = _refine(formf[...], f1w, f1b, f2w, f2b, fpre)
    rref = _refine(rolef[...], r1w, r1b, r2w, r2b, rpre)

    a1wv = a1w[...]
    le = jnp.broadcast_to(lemb[0:1], (8, 16))
    embf = _matT(le, a1wv[:, 2 * P:])
    le2 = jnp.broadcast_to(lemb[1:2], (8, 16))
    embr = _matT(le2, a1wv[:, 2 * P:])
    ztc_f = (_matT(zt, a1wv[:, :P]) + embf + a1b[...])
    ztc_r = (_matT(zt, a1wv[:, :P]) + embr + a1b[...])

    sel8 = jnp.where(lax.broadcasted_iota(jnp.int32, (8, 1), 0) == 0, 1.0, 0.0)

    def col(x8):
        return lax.dot_general(x8, sel8, (((0,), (0,)), ((), ())),
                               preferred_element_type=jnp.float32)

    def attn_mlp_t(ref_, ztc):
        a = lax.dot_general(a1wv[:, P:2 * P], ref_, (((1,), (1,)), ((), ())),
                            preferred_element_type=jnp.float32)
        a = _leaky(a + col(ztc))
        a = lax.dot_general(a2w[...], a, (((1,), (0,)), ((), ())),
                            preferred_element_type=jnp.float32)
        a = _leaky(a + col(jnp.broadcast_to(a2b[...], (8, 64))))
        a = lax.dot_general(a3w[...], a, (((1,), (0,)), ((), ())),
                            preferred_element_type=jnp.float32)
        return a + a3b[...]

    lgf = (jnp.reshape(attn_mlp_t(fref, ztc_f), (N,))
           + jnp.log(jnp.clip(fdw[...], 1e-12)))
    lgr = (jnp.reshape(attn_mlp_t(rref, ztc_r), (N,))
           + jnp.log(jnp.clip(rdw[...], 1e-12)))

    def smax(lg):
        e = jnp.exp(lg - jnp.max(lg))
        return e / jnp.sum(e)

    af = smax(lgf)
    ar = smax(lgr)
    at2_o[0:N] = af
    at2_o[N:2 * N] = ar
    af_row = jnp.reshape(af, (1, N))
    ar_row = jnp.reshape(ar, (1, N))
    df_o[...] = lax.dot_general(af_row, fref, (((1,), (0,)), ((), ())),
                                preferred_element_type=jnp.float32)
    dr_o[...] = lax.dot_general(ar_row, rref, (((1,), (0,)), ((), ())),
                                preferred_element_type=jnp.float32)
    sums_o[...] = jnp.concatenate(
        [jnp.sum(af_row, axis=1, keepdims=True),
         jnp.sum(ar_row, axis=1, keepdims=True)], axis=1)


UN = 5


def _sc_body(ei, sc, nb, attn, ekq,
             r_out,
             nb_v, fn_v, ekq_v, attn_v, S_v,
             acc_v, tmp_v, g_all, d_all, sc_all, shift_v, sem_a, sem_b,
             slots_sh, A_sh):
    c = lax.axis_index("c")
    s = lax.axis_index("s")
    base_g = s * GCH
    base_e = s * ET
    r_v = S_v

    copies_a = [
        (ekq.at[pl.ds(0, N)], ekq_v.at[pl.ds(0, N)]),
        (nb.at[pl.ds(c * N, N)], nb_v.at[pl.ds(0, N)]),
        (attn.at[pl.ds(c * N, N)], attn_v.at[pl.ds(0, N)]),
    ]
    copies_b = [
        (ei.at[pl.ds(c * 2 * E + base_e, ET)], g_all),
        (ei.at[pl.ds(c * 2 * E + E + base_e, ET)], d_all),
        (sc.at[pl.ds(c * E + base_e, ET)], sc_all),
    ]
    descs_a = [pltpu.async_copy(s_, d_, sem_a) for s_, d_ in copies_a]
    descs_b = [pltpu.async_copy(s_, d_, sem_b) for s_, d_ in copies_b]

    zero16 = jnp.zeros((16,), jnp.float32)
    big16 = jnp.full((16,), BIG, jnp.int32)

    def init_body(i, _):
        for u in range(8):
            fn_v[pl.ds(i * 128 + u * 16, 16)] = big16
            S_v[pl.ds(i * 128 + u * 16, 16)] = zero16
        return 0
    lax.fori_loop(0, NP // 128, init_body, 0)

    for u in range(UN):
        shift_v[pl.ds(u * 32, 16)] = jnp.full((16,), -1, jnp.int32)
    lane = lax.iota(jnp.int32, 16)

    for d_ in descs_a:
        d_.wait()

    def fn_body(k, _):
        for u in range(UN):
            b = (N // 16 - 1) - (k * UN + u)
            base = b * 16
            g16 = nb_v[pl.ds(base, 16)]
            key = g16 * 16 + lane
            skey = lax.sort(key, dimension=0)
            gs = lax.shift_right_logical(skey, 4)
            is16 = base + (skey & 15)
            shift_v[pl.ds(u * 32 + 1, 16)] = gs
            prev = shift_v[pl.ds(u * 32, 16)]
            mask = gs != prev
            plsc.store_scatter(fn_v, [gs], is16, mask=mask)
        return 0
    lax.fori_loop(0, N // 16 // UN, fn_body, 0)

    for d_ in descs_b:
        d_.wait()

    def s_group(j, _):
        for u in range(UN):
            off = j * (16 * UN) + u * 16
            g16 = g_all[pl.ds(off, 16)]
            d16 = d_all[pl.ds(off, 16)]
            ek = plsc.load_gather(ekq_v, [d16])
            plsc.addupdate_scatter(S_v, [g16], ek)
        return 0
    lax.fori_loop(0, ET // (16 * UN), s_group, 0)

    pltpu.sync_copy(S_v, slots_sh.at[s])

    def zero_body(i, _):
        for u in range(8):
            r_v[pl.ds(i * 128 + u * 16, 16)] = zero16
        return 0
    lax.fori_loop(0, NP // 128, zero_body, 0)

    plsc.subcore_barrier()
    pltpu.sync_copy(slots_sh.at[0, pl.ds(base_g, GCH)], acc_v)
    for t2 in range(1, 16):
        pltpu.sync_copy(slots_sh.at[t2, pl.ds(base_g, GCH)], tmp_v)

        def addb(j, _):
            for u in range(8):
                o = j * 128 + u * 16
                acc_v[pl.ds(o, 16)] = acc_v[pl.ds(o, 16)] + tmp_v[pl.ds(o, 16)]
            return 0
        lax.fori_loop(0, GCH // 128, addb, 0)

    def abody(j, _):
        for u in range(8):
            o = j * 128 + u * 16
            fn16 = fn_v[pl.ds(base_g + o, 16)]
            valid = fn16 < N
            fnc = jnp.minimum(fn16, N - 1)
            a16 = plsc.load_gather(attn_v, [fnc])
            s16 = acc_v[pl.ds(o, 16)]
            tmp_v[pl.ds(o, 16)] = jnp.where(valid, a16 / (s16 + 1e-12), 0.0)
        return 0
    lax.fori_loop(0, GCH // 128, abody, 0)
    pltpu.sync_copy(tmp_v, A_sh.at[pl.ds(base_g, GCH)])
    plsc.subcore_barrier()
    A_v = attn_v
    pltpu.sync_copy(A_sh, A_v)

    def r_group(j, _):
        for u in range(UN):
            off = j * (16 * UN) + u * 16
            g16 = g_all[pl.ds(off, 16)]
            d16 = d_all[pl.ds(off, 16)]
            s16 = sc_all[pl.ds(off, 16)]
            a = plsc.load_gather(A_v, [g16])
            ek = plsc.load_gather(ekq_v, [d16])
            plsc.addupdate_scatter(r_v, [d16], a * ek * s16)
        return 0
    lax.fori_loop(0, ET // (16 * UN), r_group, 0)

    pltpu.sync_copy(r_v, slots_sh.at[s])
    plsc.subcore_barrier()
    pltpu.sync_copy(slots_sh.at[0, pl.ds(base_g, GCH)], acc_v)
    for t2 in range(1, 16):
        pltpu.sync_copy(slots_sh.at[t2, pl.ds(base_g, GCH)], tmp_v)

        def addb2(j, _):
            for u in range(8):
                o = j * 128 + u * 16
                acc_v[pl.ds(o, 16)] = acc_v[pl.ds(o, 16)] + tmp_v[pl.ds(o, 16)]
            return 0
        lax.fori_loop(0, GCH // 128, addb2, 0)
    pltpu.sync_copy(acc_v, r_out.at[pl.ds(c * NP + base_g, GCH)])


_SC_SCRATCH = [
        pltpu.VMEM((NP,), jnp.int32),
        pltpu.VMEM((NP,), jnp.int32),
        pltpu.VMEM((NP,), jnp.float32),
        pltpu.VMEM((NP,), jnp.float32),
        pltpu.VMEM((NP,), jnp.float32),
        pltpu.VMEM((GCH,), jnp.float32),
        pltpu.VMEM((GCH,), jnp.float32),
        pltpu.VMEM((ET,), jnp.int32),
        pltpu.VMEM((ET,), jnp.int32),
        pltpu.VMEM((ET,), jnp.float32),
        pltpu.VMEM((UN * 32,), jnp.int32),
        pltpu.SemaphoreType.DMA,
        pltpu.SemaphoreType.DMA,
        pltpu.VMEM_SHARED((16, NP), jnp.float32),
        pltpu.VMEM_SHARED((NP,), jnp.float32),
    ]


@functools.lru_cache(maxsize=None)
def _get_sc_edges():
    return functools.partial(
        pl.kernel,
        out_type=jax.ShapeDtypeStruct((2 * NP,), jnp.float32),
        mesh=plsc.VectorSubcoreMesh(core_axis_name="c", subcore_axis_name="s",
                                    num_cores=2, num_subcores=16),
        scratch_types=_SC_SCRATCH,
        compiler_params=pltpu.CompilerParams(needs_layout_passes=False),
    )(_sc_body)


def _tcb_body(rall, V, zt, df, dr, sums, trust8,
              m1w, m1b, m2w, m2b,
              i1w, i1b, i2w, i2b, normw, normb,
              mixpre, integpre, dgate,
              zref_o, vprior_o, dmean_o):
    r2 = jnp.reshape(rall[...], (1, 2 * NP))
    rf = jnp.broadcast_to(r2[:, 0:N], (8, N))
    rr = jnp.broadcast_to(r2[:, NP:NP + N], (8, N))
    Vv = V[...]
    vf = lax.dot_general(rf, Vv, (((1,), (0,)), ((), ())),
                         preferred_element_type=jnp.float32)
    vr = lax.dot_general(rr, Vv, (((1,), (0,)), ((), ())),
                         preferred_element_type=jnp.float32)
    m1wv = m1w[...]
    h = (_matT(vf, m1wv[:, :D]) + _matT(vr, m1wv[:, D:2 * D])
         + _matT(trust8[...], m1wv[:, 2 * D:]) + m1b[...])
    h = _prelu(h, mixpre[...])
    l2 = _matT(h, m2w[...]) + m2b[...]
    m = jnp.max(l2, axis=1, keepdims=True)
    e2 = jnp.exp(l2 - m)
    lw = e2 / jnp.sum(e2, axis=1, keepdims=True)
    wf = lw[:, 0:1]
    wr = lw[:, 1:2]
    vprior = wf * vf + wr * vr
    zt8 = jnp.broadcast_to(zt[...], (8, P))
    fdelta = zt8 * sums[0:1, 0:1] - jnp.broadcast_to(df[...], (8, P))
    rdelta = zt8 * sums[0:1, 1:2] - jnp.broadcast_to(dr[...], (8, P))
    draw = wf * fdelta + wr * rdelta
    dmean = dgate[...] * _lnorm(draw)
    h2 = _prelu(_matT(vprior, i1w[...]) + i1b[...], integpre[...])
    zref = (_lnorm(zt8 + _matT(h2, i2w[...]) + i2b[...])
            * normw[...] + normb[...])
    zref_o[...] = zref[0:1]
    vprior_o[...] = vprior[0:1]
    dmean_o[...] = dmean[0:1]


def kernel(params, target_features, form_features, role_features,
           form_neighbors, form_binds_ei, form_binds_y, form_binds_w,
           role_neighbors, role_binds_ei, role_binds_y, role_binds_w,
           form_diff_w, role_diff_w, trust_vector, drug_features):
    p = params
    f32 = jnp.float32

    t8 = jnp.tile(target_features[None, :].astype(f32), (8, 1))

    refine_ws = (
        p['form1_w'], p['form1_b'][None], p['form2_w'], p['form2_b'][None],
        p['role1_w'], p['role1_b'][None], p['role2_w'], p['role2_b'][None],
    )
    preluses = (p['form_prelu'][None, None], p['role_prelu'][None, None])

    zt, ekq, V, sc2 = pl.pallas_call(
        _tca1_body,
        out_shape=[
            jax.ShapeDtypeStruct((1, P), f32),
            jax.ShapeDtypeStruct((NDRUG,), f32),
            jax.ShapeDtypeStruct((NDRUG, D), f32),
            jax.ShapeDtypeStruct((2 * E,), f32),
        ],
    )(
        t8, drug_features,
        form_binds_y, form_binds_w, role_binds_y, role_binds_w,
        *refine_ws, *preluses,
        p['q_w'], p['q_b'][None], p['k_w'], p['k_b'][None],
        p['v_w'], p['v_b'][None],
    )

    at2, df, dr, sums = pl.pallas_call(
        _tca2_body,
        out_shape=[
            jax.ShapeDtypeStruct((2 * N,), f32),
            jax.ShapeDtypeStruct((1, P), f32),
            jax.ShapeDtypeStruct((1, P), f32),
            jax.ShapeDtypeStruct((1, 2), f32),
        ],
    )(
        t8, form_features, role_features, form_diff_w, role_diff_w,
        p['layer_emb'],
        *refine_ws,
        p['attn1_w'], p['attn1_b'][None],
        p['attn2_w'], p['attn2_b'][None], p['attn3_w'], p['attn3_b'][None],
        *preluses,
    )

    ei_all = jnp.concatenate([form_binds_ei.reshape(-1),
                              role_binds_ei.reshape(-1)]).astype(jnp.int32)
    nb_all = jnp.concatenate([form_neighbors,
                              role_neighbors]).astype(jnp.int32)
    r = _get_sc_edges()(ei_all, sc2, nb_all, at2, ekq)

    zref, vprior, dmean = pl.pallas_call(
        _tcb_body,
        out_shape=[
            jax.ShapeDtypeStruct((1, P), f32),
            jax.ShapeDtypeStruct((1, P), f32),
            jax.ShapeDtypeStruct((1, P), f32),
        ],
    )(
        r, V, zt, df, dr, sums,
        jnp.tile(trust_vector[None, :].astype(f32), (8, 1)),
        p['mix1_w'], p['mix1_b'][None],
        p['mix2_w'], p['mix2_b'][None],
        p['integ1_w'], p['integ1_b'][None],
        p['integ2_w'], p['integ2_b'][None],
        p['norm_w'][None], p['norm_b'][None],
        p['mix_prelu'][None, None], p['integ_prelu'][None, None],
        p['delta_gate'][None, None],
    )

    return (zref[0], vprior[0], dmean[0], at2[:N], at2[N:])
